# baseline (speedup 1.0000x reference)
import functools

import jax
import jax.numpy as jnp
from jax import lax
from jax.experimental import pallas as pl
from jax.experimental.pallas import tpu as pltpu
from jax.experimental.pallas import tpu_sc as plsc

N = 10000
NPAD = 10240
D = 128
NC = 2
NS = 16
NW = NC * NS
CHUNK = 128
RPW = NPAD // NS
RB = 640
PADROW = N + 8

_MESH = plsc.VectorSubcoreMesh(core_axis_name="c", subcore_axis_name="s")
_F32 = jnp.float32

A0 = 152
A1 = 8
G0 = 88
G1 = 72
AW = 8
KT = A0 + A1
assert A0 % AW == 0 and A1 <= AW
assert G0 + G1 == KT


def _sc_degree(dst2, z128, ones):
    @functools.partial(
        pl.kernel,
        out_type=jax.ShapeDtypeStruct((NC, NPAD, D), _F32),
        mesh=_MESH,
        scratch_types=[
            pltpu.VMEM((max(G0, G1), CHUNK), jnp.int32),
            pltpu.VMEM((CHUNK, D), _F32),
            pltpu.VMEM_SHARED((NPAD, D), _F32),
        ],
    )
    def k(dst_hbm, z_hbm, ones_hbm, out_hbm, di_v, ones_v, deg_sh):
        c = lax.axis_index("c")
        s = lax.axis_index("s")
        pltpu.sync_copy(z_hbm.at[pl.ds(s * RPW, RPW)],
                        deg_sh.at[pl.ds(s * RPW, RPW)])
        pltpu.sync_copy(ones_hbm, ones_v)

        def work(nk, base):
            pltpu.sync_copy(dst_hbm.at[pl.ds(base, nk)], di_v.at[pl.ds(0, nk)])
            plsc.subcore_barrier()

            @pl.loop(0, nk)
            def _(i):
                pltpu.sync_copy(ones_v, deg_sh.at[di_v.at[i]], add=True)

        @pl.when(c == 0)
        def _():
            work(G0, s * G0)

        @pl.when(c == 1)
        def _():
            work(G1, NS * G0 + s * G1)

        plsc.subcore_barrier()
        pltpu.sync_copy(deg_sh.at[pl.ds(s * RPW, RPW)],
                        out_hbm.at[c].at[pl.ds(s * RPW, RPW)])

    return k(dst2, z128, ones)


def _sc_aggregate(y, src2, dst2, z128):
    @functools.partial(
        pl.kernel,
        out_type=jax.ShapeDtypeStruct((NC, NPAD, D), _F32),
        mesh=_MESH,
        scratch_types=[
            pltpu.VMEM((AW, CHUNK), jnp.int32),
            pltpu.VMEM((AW, CHUNK), jnp.int32),
            pltpu.VMEM((CHUNK, D), _F32),
            pltpu.VMEM((CHUNK, D), _F32),
            pltpu.VMEM_SHARED((NPAD, D), _F32),
            pltpu.SemaphoreType.DMA,
            pltpu.SemaphoreType.DMA,
        ],
    )
    def k(y_hbm, src_hbm, dst_hbm, z_hbm, out_hbm,
          si_v, di_v, buf_a, buf_b, acc_sh, sem_a, sem_b):
        c = lax.axis_index("c")
        s = lax.axis_index("s")
        pltpu.sync_copy(z_hbm.at[pl.ds(s * RPW, RPW)],
                        acc_sh.at[pl.ds(s * RPW, RPW)])

        def window(win, wbase, first):
            pltpu.sync_copy(src_hbm.at[pl.ds(wbase, win)], si_v.at[pl.ds(0, win)])
            pltpu.sync_copy(dst_hbm.at[pl.ds(wbase, win)], di_v.at[pl.ds(0, win)])
            if first:
                plsc.subcore_barrier()
            pltpu.async_copy(y_hbm.at[si_v.at[0]], buf_a, sem_a)

            @pl.loop(0, win // 2 - 1)
            def _(j):
                i = 2 * j
                pltpu.make_async_copy(y_hbm.at[si_v.at[i]], buf_a, sem_a).wait()
                pltpu.async_copy(y_hbm.at[si_v.at[i + 1]], buf_b, sem_b)
                pltpu.sync_copy(buf_a, acc_sh.at[di_v.at[i]], add=True)
                pltpu.make_async_copy(y_hbm.at[si_v.at[i + 1]], buf_b, sem_b).wait()
                pltpu.async_copy(y_hbm.at[si_v.at[i + 2]], buf_a, sem_a)
                pltpu.sync_copy(buf_b, acc_sh.at[di_v.at[i + 1]], add=True)

            i = win - 2
            pltpu.make_async_copy(y_hbm.at[si_v.at[i]], buf_a, sem_a).wait()
            pltpu.async_copy(y_hbm.at[si_v.at[i + 1]], buf_b, sem_b)
            pltpu.sync_copy(buf_a, acc_sh.at[di_v.at[i]], add=True)
            pltpu.make_async_copy(y_hbm.at[si_v.at[i + 1]], buf_b, sem_b).wait()
            pltpu.sync_copy(buf_b, acc_sh.at[di_v.at[i + 1]], add=True)

        def work(nk, win, base):
            for wi in range(nk // win):
                window(win, base + wi * win, wi == 0)

        @pl.when(c == 0)
        def _():
            work(A0, AW, s * A0)

        @pl.when(c == 1)
        def _():
            work(A1, A1, NS * A0 + s * A1)

        plsc.subcore_barrier()
        pltpu.sync_copy(acc_sh.at[pl.ds(s * RPW, RPW)],
                        out_hbm.at[c].at[pl.ds(s * RPW, RPW)])

    return k(y, src2, dst2, z128)


def _dinv_block(d_ref):
    return lax.rsqrt(d_ref[0, :, 0:1] + d_ref[1, :, 0:1] + 1.0)


_DEG_SPEC = pl.BlockSpec((NC, RB, D), lambda i: (0, i, 0))


def _tc_y1(x32, W1, degp):
    def body(x_ref, w_ref, d_ref, o_ref):
        xw = jnp.dot(x_ref[...], w_ref[...],
                     preferred_element_type=_F32,
                     precision=lax.Precision.HIGHEST)
        o_ref[...] = xw * _dinv_block(d_ref)

    return pl.pallas_call(
        body,
        grid=(NPAD // RB,),
        in_specs=[
            pl.BlockSpec((RB, D), lambda i: (i, 0)),
            pl.BlockSpec((D, D), lambda i: (0, 0)),
            _DEG_SPEC,
        ],
        out_specs=pl.BlockSpec((RB, D), lambda i: (i, 0)),
        out_shape=jax.ShapeDtypeStruct((NPAD, D), _F32),
    )(x32, W1, degp)


def _tc_mid(p, y1, degp, b1, W2):
    def body(p_ref, y_ref, d_ref, b_ref, w_ref, o_ref):
        dinv = _dinv_block(d_ref)
        h = (p_ref[0] + p_ref[1] + y_ref[...]) * dinv + b_ref[...]
        h = jnp.maximum(h, 0.0)
        o_ref[...] = jnp.dot(h, w_ref[...],
                             preferred_element_type=_F32,
                             precision=lax.Precision.HIGHEST) * dinv

    return pl.pallas_call(
        body,
        grid=(NPAD // RB,),
        in_specs=[
            pl.BlockSpec((NC, RB, D), lambda i: (0, i, 0)),
            pl.BlockSpec((RB, D), lambda i: (i, 0)),
            _DEG_SPEC,
            pl.BlockSpec((1, D), lambda i: (0, 0)),
            pl.BlockSpec((D, D), lambda i: (0, 0)),
        ],
        out_specs=pl.BlockSpec((RB, D), lambda i: (i, 0)),
        out_shape=jax.ShapeDtypeStruct((NPAD, D), _F32),
    )(p, y1, degp, b1, W2)


def _tc_fin(q, y2, degp, b2):
    def body(q_ref, y_ref, d_ref, b_ref, o_ref):
        o_ref[...] = ((q_ref[0] + q_ref[1] + y_ref[...]) * _dinv_block(d_ref)
                      + b_ref[...])

    return pl.pallas_call(
        body,
        grid=(NPAD // RB,),
        in_specs=[
            pl.BlockSpec((NC, RB, D), lambda i: (0, i, 0)),
            pl.BlockSpec((RB, D), lambda i: (i, 0)),
            _DEG_SPEC,
            pl.BlockSpec((1, D), lambda i: (0, 0)),
        ],
        out_specs=pl.BlockSpec((RB, D), lambda i: (i, 0)),
        out_shape=jax.ShapeDtypeStruct((NPAD, D), _F32),
    )(q, y2, degp, b2)


def kernel(x, edge_index, W1, b1, W2, b2):
    e = edge_index.shape[1]
    epad = NS * KT * CHUNK
    assert e <= epad

    src = edge_index[0].astype(jnp.int32)
    dst = edge_index[1].astype(jnp.int32)
    padv = jnp.full((epad - e,), PADROW, jnp.int32)
    src2 = jnp.concatenate([src, padv]).reshape(NS * KT, CHUNK)
    dst2 = jnp.concatenate([dst, padv]).reshape(NS * KT, CHUNK)

    x32 = jnp.zeros((NPAD, D), _F32).at[:N].set(x.astype(_F32))
    z128 = jnp.zeros((NPAD, D), _F32)
    ones = jnp.ones((CHUNK, D), _F32)

    degp = _sc_degree(dst2, z128, ones)
    y1 = _tc_y1(x32, W1, degp)
    p = _sc_aggregate(y1, src2, dst2, z128)
    y2 = _tc_mid(p, y1, degp, b1.reshape(1, D), W2)
    q = _sc_aggregate(y2, src2, dst2, z128)
    out = _tc_fin(q, y2, degp, b2.reshape(1, D))
    return out[:N]


# speedup vs baseline: 1.0308x; 1.0308x over previous
import functools

import jax
import jax.numpy as jnp
from jax import lax
from jax.experimental import pallas as pl
from jax.experimental.pallas import tpu as pltpu
from jax.experimental.pallas import tpu_sc as plsc

N = 10000
NPAD = 10240
D = 128
NC = 2
NS = 16
NW = NC * NS
CHUNK = 128
RPW = NPAD // NS
RB = 640
PADROW = N + 8

_MESH = plsc.VectorSubcoreMesh(core_axis_name="c", subcore_axis_name="s")
_F32 = jnp.float32

A0 = 144
A1 = 16
G0 = 88
G1 = 72
AW = 48
KT = A0 + A1
assert A0 % AW == 0 and A1 <= AW
assert G0 + G1 == KT


def _sc_degree(dst2, z128, ones):
    @functools.partial(
        pl.kernel,
        out_type=jax.ShapeDtypeStruct((NC, NPAD, D), _F32),
        mesh=_MESH,
        scratch_types=[
            pltpu.VMEM((max(G0, G1), CHUNK), jnp.int32),
            pltpu.VMEM((CHUNK, D), _F32),
            pltpu.VMEM_SHARED((NPAD, D), _F32),
        ],
    )
    def k(dst_hbm, z_hbm, ones_hbm, out_hbm, di_v, ones_v, deg_sh):
        c = lax.axis_index("c")
        s = lax.axis_index("s")
        pltpu.sync_copy(z_hbm.at[pl.ds(s * RPW, RPW)],
                        deg_sh.at[pl.ds(s * RPW, RPW)])
        pltpu.sync_copy(ones_hbm, ones_v)

        def work(nk, base):
            pltpu.sync_copy(dst_hbm.at[pl.ds(base, nk)], di_v.at[pl.ds(0, nk)])
            plsc.subcore_barrier()

            @pl.loop(0, nk)
            def _(i):
                pltpu.sync_copy(ones_v, deg_sh.at[di_v.at[i]], add=True)

        @pl.when(c == 0)
        def _():
            work(G0, s * G0)

        @pl.when(c == 1)
        def _():
            work(G1, NS * G0 + s * G1)

        plsc.subcore_barrier()
        pltpu.sync_copy(deg_sh.at[pl.ds(s * RPW, RPW)],
                        out_hbm.at[c].at[pl.ds(s * RPW, RPW)])

    return k(dst2, z128, ones)


def _sc_aggregate(y, src2, dst2, z128):
    @functools.partial(
        pl.kernel,
        out_type=jax.ShapeDtypeStruct((NC, NPAD, D), _F32),
        mesh=_MESH,
        scratch_types=[
            pltpu.VMEM((AW, CHUNK), jnp.int32),
            pltpu.VMEM((AW, CHUNK), jnp.int32),
            pltpu.VMEM((CHUNK, D), _F32),
            pltpu.VMEM((CHUNK, D), _F32),
            pltpu.VMEM_SHARED((NPAD, D), _F32),
            pltpu.SemaphoreType.DMA,
            pltpu.SemaphoreType.DMA,
        ],
    )
    def k(y_hbm, src_hbm, dst_hbm, z_hbm, out_hbm,
          si_v, di_v, buf_a, buf_b, acc_sh, sem_a, sem_b):
        c = lax.axis_index("c")
        s = lax.axis_index("s")
        pltpu.sync_copy(z_hbm.at[pl.ds(s * RPW, RPW)],
                        acc_sh.at[pl.ds(s * RPW, RPW)])

        def window(win, wbase, first):
            pltpu.sync_copy(src_hbm.at[pl.ds(wbase, win)], si_v.at[pl.ds(0, win)])
            pltpu.sync_copy(dst_hbm.at[pl.ds(wbase, win)], di_v.at[pl.ds(0, win)])
            if first:
                plsc.subcore_barrier()
            pltpu.async_copy(y_hbm.at[si_v.at[0]], buf_a, sem_a)

            @pl.loop(0, win // 2 - 1)
            def _(j):
                i = 2 * j
                pltpu.make_async_copy(y_hbm.at[si_v.at[i]], buf_a, sem_a).wait()
                pltpu.async_copy(y_hbm.at[si_v.at[i + 1]], buf_b, sem_b)
                pltpu.sync_copy(buf_a, acc_sh.at[di_v.at[i]], add=True)
                pltpu.make_async_copy(y_hbm.at[si_v.at[i + 1]], buf_b, sem_b).wait()
                pltpu.async_copy(y_hbm.at[si_v.at[i + 2]], buf_a, sem_a)
                pltpu.sync_copy(buf_b, acc_sh.at[di_v.at[i + 1]], add=True)

            i = win - 2
            pltpu.make_async_copy(y_hbm.at[si_v.at[i]], buf_a, sem_a).wait()
            pltpu.async_copy(y_hbm.at[si_v.at[i + 1]], buf_b, sem_b)
            pltpu.sync_copy(buf_a, acc_sh.at[di_v.at[i]], add=True)
            pltpu.make_async_copy(y_hbm.at[si_v.at[i + 1]], buf_b, sem_b).wait()
            pltpu.sync_copy(buf_b, acc_sh.at[di_v.at[i + 1]], add=True)

        def work(nk, win, base):
            for wi in range(nk // win):
                window(win, base + wi * win, wi == 0)

        @pl.when(c == 0)
        def _():
            work(A0, AW, s * A0)

        @pl.when(c == 1)
        def _():
            work(A1, A1, NS * A0 + s * A1)

        plsc.subcore_barrier()
        pltpu.sync_copy(acc_sh.at[pl.ds(s * RPW, RPW)],
                        out_hbm.at[c].at[pl.ds(s * RPW, RPW)])

    return k(y, src2, dst2, z128)


def _dinv_block(d_ref):
    return lax.rsqrt(d_ref[0, :, 0:1] + d_ref[1, :, 0:1] + 1.0)


_DEG_SPEC = pl.BlockSpec((NC, RB, D), lambda i: (0, i, 0))


def _tc_y1(x32, W1, degp):
    def body(x_ref, w_ref, d_ref, o_ref):
        xw = jnp.dot(x_ref[...], w_ref[...],
                     preferred_element_type=_F32,
                     precision=lax.Precision.HIGHEST)
        o_ref[...] = xw * _dinv_block(d_ref)

    return pl.pallas_call(
        body,
        grid=(NPAD // RB,),
        in_specs=[
            pl.BlockSpec((RB, D), lambda i: (i, 0)),
            pl.BlockSpec((D, D), lambda i: (0, 0)),
            _DEG_SPEC,
        ],
        out_specs=pl.BlockSpec((RB, D), lambda i: (i, 0)),
        out_shape=jax.ShapeDtypeStruct((NPAD, D), _F32),
    )(x32, W1, degp)


def _tc_mid(p, y1, degp, b1, W2):
    def body(p_ref, y_ref, d_ref, b_ref, w_ref, o_ref):
        dinv = _dinv_block(d_ref)
        h = (p_ref[0] + p_ref[1] + y_ref[...]) * dinv + b_ref[...]
        h = jnp.maximum(h, 0.0)
        o_ref[...] = jnp.dot(h, w_ref[...],
                             preferred_element_type=_F32,
                             precision=lax.Precision.HIGHEST) * dinv

    return pl.pallas_call(
        body,
        grid=(NPAD // RB,),
        in_specs=[
            pl.BlockSpec((NC, RB, D), lambda i: (0, i, 0)),
            pl.BlockSpec((RB, D), lambda i: (i, 0)),
            _DEG_SPEC,
            pl.BlockSpec((1, D), lambda i: (0, 0)),
            pl.BlockSpec((D, D), lambda i: (0, 0)),
        ],
        out_specs=pl.BlockSpec((RB, D), lambda i: (i, 0)),
        out_shape=jax.ShapeDtypeStruct((NPAD, D), _F32),
    )(p, y1, degp, b1, W2)


def _tc_fin(q, y2, degp, b2):
    def body(q_ref, y_ref, d_ref, b_ref, o_ref):
        o_ref[...] = ((q_ref[0] + q_ref[1] + y_ref[...]) * _dinv_block(d_ref)
                      + b_ref[...])

    return pl.pallas_call(
        body,
        grid=(NPAD // RB,),
        in_specs=[
            pl.BlockSpec((NC, RB, D), lambda i: (0, i, 0)),
            pl.BlockSpec((RB, D), lambda i: (i, 0)),
            _DEG_SPEC,
            pl.BlockSpec((1, D), lambda i: (0, 0)),
        ],
        out_specs=pl.BlockSpec((RB, D), lambda i: (i, 0)),
        out_shape=jax.ShapeDtypeStruct((NPAD, D), _F32),
    )(q, y2, degp, b2)


def kernel(x, edge_index, W1, b1, W2, b2):
    e = edge_index.shape[1]
    epad = NS * KT * CHUNK
    assert e <= epad

    src = edge_index[0].astype(jnp.int32)
    dst = edge_index[1].astype(jnp.int32)
    padv = jnp.full((epad - e,), PADROW, jnp.int32)
    src2 = jnp.concatenate([src, padv]).reshape(NS * KT, CHUNK)
    dst2 = jnp.concatenate([dst, padv]).reshape(NS * KT, CHUNK)

    x32 = jnp.zeros((NPAD, D), _F32).at[:N].set(x.astype(_F32))
    z128 = jnp.zeros((NPAD, D), _F32)
    ones = jnp.ones((CHUNK, D), _F32)

    degp = _sc_degree(dst2, z128, ones)
    y1 = _tc_y1(x32, W1, degp)
    p = _sc_aggregate(y1, src2, dst2, z128)
    y2 = _tc_mid(p, y1, degp, b1.reshape(1, D), W2)
    q = _sc_aggregate(y2, src2, dst2, z128)
    out = _tc_fin(q, y2, degp, b2.reshape(1, D))
    return out[:N]
